# baseline (speedup 1.0000x reference)
import functools
import math

import jax
import jax.numpy as jnp
from jax import lax
from jax.experimental import pallas as pl
from jax.experimental.pallas import tpu as pltpu
from jax.experimental.pallas import tpu_sc as plsc

D = 128
S = 200
B = 1024
N = B * S
NC = 2
NS = 16
NW = NC * NS
PER_W = N // NW
R = 128
CHUNKS = PER_W // R
PAIRS = CHUNKS // 2
NSLOT = 3
LANES = 16
SCALE = math.sqrt(float(D))

_mesh = plsc.VectorSubcoreMesh(core_axis_name="c", subcore_axis_name="s")


@functools.partial(
    pl.kernel,
    mesh=_mesh,
    out_type=jax.ShapeDtypeStruct((N, D), jnp.float32),
    scratch_types=[
        pltpu.VMEM((CHUNKS, R), jnp.int32),
        pltpu.VMEM((S, D), jnp.float32),
    ]
    + [pltpu.VMEM((R, D), jnp.float32) for _ in range(2 * NSLOT)]
    + [pltpu.SemaphoreType.DMA for _ in range(4 * NSLOT)],
)
def _emb_kernel(idx_hbm, table_hbm, pe_hbm, out_hbm, idx_v, pe_v, *rest):
    bufs = rest[:2 * NSLOT]
    gsems = rest[2 * NSLOT:4 * NSLOT]
    wsems = rest[4 * NSLOT:]

    wid = lax.axis_index("s") * NC + lax.axis_index("c")
    base = wid * PER_W

    pltpu.sync_copy(idx_hbm.at[wid], idx_v)
    pltpu.sync_copy(pe_hbm.at[pl.ds(0, S)], pe_v)

    def gather(c, b):
        pltpu.async_copy(table_hbm.at[idx_v.at[c]], bufs[b], gsems[b])

    def wait_gather(c, b):
        pltpu.make_async_copy(table_hbm.at[idx_v.at[c]], bufs[b],
                              gsems[b]).wait()

    def write(c, b):
        pltpu.async_copy(bufs[b], out_hbm.at[pl.ds(base + c * R, R)],
                         wsems[b])

    def wait_write(c, b):
        pltpu.make_async_copy(bufs[b], out_hbm.at[pl.ds(base + c * R, R)],
                              wsems[b]).wait()

    for p in range(2):
        gather(p, 2 * p)
        gather(p + PAIRS, 2 * p + 1)

    def pair_step(t, slot):
        sA, sB = 2 * slot, 2 * slot + 1
        nslot = (slot + 2) % NSLOT
        nA, nB = 2 * nslot, 2 * nslot + 1

        @pl.when(t >= 1)
        def _():
            wait_write(t - 1, nA)
            wait_write(t - 1 + PAIRS, nB)

        @pl.when(t + 2 < PAIRS)
        def _():
            gather(t + 2, nA)
            gather(t + 2 + PAIRS, nB)

        wait_gather(t, sA)
        wait_gather(t + PAIRS, sB)

        pos0 = lax.rem(t * R, S)

        @plsc.parallel_loop(0, R, 1, unroll=2)
        def row_body(r):
            pos = pos0 + r
            pos = lax.select(pos >= S, pos - S, pos)
            for j in range(D // LANES):
                sl = pl.ds(j * LANES, LANES)
                pvec = pe_v[pos, sl]
                bufs[sA][r, sl] = bufs[sA][r, sl] * SCALE + pvec
                bufs[sB][r, sl] = bufs[sB][r, sl] * SCALE + pvec

        write(t, sA)
        write(t + PAIRS, sB)

    def outer(i, carry):
        for k in range(NSLOT):
            pair_step(i * NSLOT + k, k)
        return carry

    lax.fori_loop(0, (PAIRS - 1) // NSLOT, outer, 0)

    pair_step(PAIRS - 1, (PAIRS - 1) % NSLOT)
    wait_write(PAIRS - 1, 2 * ((PAIRS - 1) % NSLOT))
    wait_write(2 * PAIRS - 1, 2 * ((PAIRS - 1) % NSLOT) + 1)


def kernel(x, table, pe):
    idx = jnp.reshape(x, (NW, CHUNKS, R))
    out = _emb_kernel(idx, table, pe)
    return jnp.reshape(out, (B, S, D))


# speedup vs baseline: 1.0609x; 1.0609x over previous
import functools
import math

import jax
import jax.numpy as jnp
from jax import lax
from jax.experimental import pallas as pl
from jax.experimental.pallas import tpu as pltpu
from jax.experimental.pallas import tpu_sc as plsc

D = 128
S = 200
B = 1024
N = B * S
NC = 2
NS = 16
NW = NC * NS
PER_W = N // NW
R = 128
CHUNKS = PER_W // R
NB = 5
LANES = 16
SCALE = math.sqrt(float(D))

_mesh = plsc.VectorSubcoreMesh(core_axis_name="c", subcore_axis_name="s")


@functools.partial(
    pl.kernel,
    mesh=_mesh,
    out_type=jax.ShapeDtypeStruct((N, D), jnp.float32),
    scratch_types=[
        pltpu.VMEM((CHUNKS, R), jnp.int32),
        pltpu.VMEM((S, D), jnp.float32),
    ]
    + [pltpu.VMEM((R, D), jnp.float32) for _ in range(NB)]
    + [pltpu.SemaphoreType.DMA for _ in range(2 * NB + 1)],
)
def _emb_kernel(idx_hbm, table_hbm, pe_hbm, out_hbm, idx_v, pe_v, *rest):
    bufs = rest[:NB]
    gsems = rest[NB:2 * NB]
    wsems = rest[2 * NB:3 * NB]
    pe_sem = rest[3 * NB]

    wid = lax.axis_index("s") * NC + lax.axis_index("c")
    base = wid * PER_W

    pltpu.sync_copy(idx_hbm.at[wid], idx_v)

    def gather(c, b):
        pltpu.async_copy(table_hbm.at[idx_v.at[c]], bufs[b], gsems[b])

    def wait_gather(c, b):
        pltpu.make_async_copy(table_hbm.at[idx_v.at[c]], bufs[b],
                              gsems[b]).wait()

    def write(c, b):
        pltpu.async_copy(bufs[b], out_hbm.at[pl.ds(base + c * R, R)],
                         wsems[b])

    def wait_write(c, b):
        pltpu.make_async_copy(bufs[b], out_hbm.at[pl.ds(base + c * R, R)],
                              wsems[b]).wait()

    pltpu.async_copy(pe_hbm.at[pl.ds(0, S)], pe_v, pe_sem)
    for k in range(NB):
        gather(k, k)
    pltpu.make_async_copy(pe_hbm.at[pl.ds(0, S)], pe_v, pe_sem).wait()

    def step(c, b):
        wait_gather(c, b)

        bn = (b + 3) % NB

        @pl.when((c >= NB - 3) & (c + 3 < CHUNKS))
        def _():
            wait_write(c - (NB - 3), bn)
            gather(c + 3, bn)

        pos0 = lax.rem(c * R, S)

        @plsc.parallel_loop(0, R, 1, unroll=4)
        def row_body(r):
            pos = pos0 + r
            pos = lax.select(pos >= S, pos - S, pos)
            for j in range(D // LANES):
                sl = pl.ds(j * LANES, LANES)
                bufs[b][r, sl] = bufs[b][r, sl] * SCALE + pe_v[pos, sl]

        write(c, b)

    def outer(i, carry):
        for k in range(NB):
            step(i * NB + k, k)
        return carry

    lax.fori_loop(0, CHUNKS // NB, outer, 0)

    for k in range(NB):
        c = CHUNKS - NB + k
        wait_write(c, c % NB)


def kernel(x, table, pe):
    idx = jnp.reshape(x, (NW, CHUNKS, R))
    out = _emb_kernel(idx, table, pe)
    return jnp.reshape(out, (B, S, D))
